# Initial kernel scaffold; baseline (speedup 1.0000x reference)
import jax
import jax.numpy as jnp
from jax.experimental import pallas as pl


def kernel(z, codebook):
    raise NotImplementedError("write your pallas kernel here")


# speedup vs baseline: 2.1438x; 2.1438x over previous
import functools

import jax
import jax.numpy as jnp
from jax.experimental import pallas as pl
from jax.experimental.pallas import tpu as pltpu

_NUM_CODES = 1024
_CODE_DIM = 64
_BETA = 0.25


def _vq_kernel(z_ref, cb_ref, zq_ref, idx_ref, sums_ref,
               counts_ref, loss_ref, *, nsteps, total_px):
    b = pl.program_id(0)

    @pl.when(b == 0)
    def _init():
        counts_ref[...] = jnp.zeros_like(counts_ref)
        loss_ref[...] = jnp.zeros_like(loss_ref)

    cb = cb_ref[...]
    z_b = z_ref[0]

    e_sq = jnp.sum(cb * cb, axis=1, keepdims=True)
    z_sq = jnp.sum(z_b * z_b, axis=0, keepdims=True)
    s = jax.lax.dot_general(cb, z_b, (((1,), (0,)), ((), ())),
                            preferred_element_type=jnp.float32)
    d = (z_sq + e_sq) - 2.0 * s

    min_d = jnp.min(d, axis=0, keepdims=True)
    code_iota = jax.lax.broadcasted_iota(jnp.int32, d.shape, 0)
    idx = jnp.min(jnp.where(d == min_d, code_iota, _NUM_CODES),
                  axis=0, keepdims=True)
    idx_ref[0] = idx

    onehot = (code_iota == idx).astype(jnp.float32)
    zq_ref[0] = jax.lax.dot_general(cb, onehot, (((0,), (0,)), ((), ())),
                                    preferred_element_type=jnp.float32)

    counts_ref[...] += jnp.sum(onehot, axis=1, keepdims=True)
    loss_ref[...] += min_d

    @pl.when(b == nsteps - 1)
    def _finalize():
        loss_sum = jnp.sum(loss_ref[...], axis=1, keepdims=True)
        vq_loss = loss_sum * ((1.0 + _BETA) / total_px)
        p = counts_ref[...] * (1.0 / (total_px / _CODE_DIM))
        ent = jnp.sum(p * jnp.log(p + 1e-10), axis=0, keepdims=True)
        perplexity = jnp.exp(-ent)
        sums_ref[0:1, 0:1] = vq_loss
        sums_ref[0:1, 1:2] = perplexity


def kernel(z, codebook):
    B, C, H, W = z.shape
    P = H * W
    z_r = z.reshape(B, C, P)
    total_px = B * C * P

    grid = (B,)
    zq3, idx3, sums = pl.pallas_call(
        functools.partial(_vq_kernel, nsteps=B, total_px=total_px),
        grid=grid,
        in_specs=[
            pl.BlockSpec((1, C, P), lambda b: (b, 0, 0)),
            pl.BlockSpec((_NUM_CODES, _CODE_DIM), lambda b: (0, 0)),
        ],
        out_specs=[
            pl.BlockSpec((1, C, P), lambda b: (b, 0, 0)),
            pl.BlockSpec((1, 1, P), lambda b: (b, 0, 0)),
            pl.BlockSpec((8, 128), lambda b: (0, 0)),
        ],
        out_shape=[
            jax.ShapeDtypeStruct((B, C, P), jnp.float32),
            jax.ShapeDtypeStruct((B, 1, P), jnp.int32),
            jax.ShapeDtypeStruct((8, 128), jnp.float32),
        ],
        scratch_shapes=[
            pltpu.VMEM((_NUM_CODES, 1), jnp.float32),
            pltpu.VMEM((1, P), jnp.float32),
        ],
    )(z_r, codebook)

    z_q_st = zq3.reshape(B, C, H, W)
    indices = idx3.reshape(B, H, W)
    return (z_q_st, indices, sums[0, 0], sums[0, 1])


# speedup vs baseline: 2.1693x; 1.0119x over previous
import functools

import jax
import jax.numpy as jnp
from jax.experimental import pallas as pl
from jax.experimental.pallas import tpu as pltpu

_NUM_CODES = 1024
_CODE_DIM = 64
_BETA = 0.25


def _vq_kernel(z_ref, cb_ref, zq_ref, idx_ref, sums_ref,
               counts_ref, loss_ref, *, nsteps, total_px):
    b = pl.program_id(0)

    @pl.when(b == 0)
    def _init():
        counts_ref[...] = jnp.zeros_like(counts_ref)
        loss_ref[...] = jnp.zeros_like(loss_ref)

    cb = cb_ref[...]
    z_b = z_ref[0]

    e_sq = jnp.sum(cb * cb, axis=1, keepdims=True)
    z_sq = jnp.sum(z_b * z_b, axis=0, keepdims=True)
    s = jax.lax.dot_general(cb * -2.0, z_b, (((1,), (0,)), ((), ())),
                            preferred_element_type=jnp.float32)
    d = (z_sq + e_sq) + s

    min_d = jnp.min(d, axis=0, keepdims=True)
    code_iota = jax.lax.broadcasted_iota(jnp.int32, d.shape, 0)
    idx = jnp.min(jnp.where(d == min_d, code_iota, _NUM_CODES),
                  axis=0, keepdims=True)
    idx_ref[0] = idx

    onehot = (code_iota == idx).astype(jnp.float32)
    zq_ref[0] = jax.lax.dot_general(cb, onehot, (((0,), (0,)), ((), ())),
                                    preferred_element_type=jnp.float32)

    counts_ref[...] += jnp.sum(onehot, axis=1, keepdims=True)
    loss_ref[...] += min_d

    @pl.when(b == nsteps - 1)
    def _finalize():
        loss_sum = jnp.sum(loss_ref[...], axis=1, keepdims=True)
        vq_loss = loss_sum * ((1.0 + _BETA) / total_px)
        p = counts_ref[...] * (1.0 / (total_px / _CODE_DIM))
        ent = jnp.sum(p * jnp.log(p + 1e-10), axis=0, keepdims=True)
        perplexity = jnp.exp(-ent)
        sums_ref[0:1, 0:1] = vq_loss
        sums_ref[0:1, 1:2] = perplexity


def kernel(z, codebook):
    B, C, H, W = z.shape
    P = H * W
    z_r = z.reshape(B, C, P)
    total_px = B * C * P

    grid = (B,)
    zq3, idx3, sums = pl.pallas_call(
        functools.partial(_vq_kernel, nsteps=B, total_px=total_px),
        grid=grid,
        in_specs=[
            pl.BlockSpec((1, C, P), lambda b: (b, 0, 0)),
            pl.BlockSpec((_NUM_CODES, _CODE_DIM), lambda b: (0, 0)),
        ],
        out_specs=[
            pl.BlockSpec((1, C, P), lambda b: (b, 0, 0)),
            pl.BlockSpec((1, 1, P), lambda b: (b, 0, 0)),
            pl.BlockSpec((8, 128), lambda b: (0, 0)),
        ],
        out_shape=[
            jax.ShapeDtypeStruct((B, C, P), jnp.float32),
            jax.ShapeDtypeStruct((B, 1, P), jnp.int32),
            jax.ShapeDtypeStruct((8, 128), jnp.float32),
        ],
        scratch_shapes=[
            pltpu.VMEM((_NUM_CODES, 1), jnp.float32),
            pltpu.VMEM((1, P), jnp.float32),
        ],
    )(z_r, codebook)

    z_q_st = zq3.reshape(B, C, H, W)
    indices = idx3.reshape(B, H, W)
    return (z_q_st, indices, sums[0, 0], sums[0, 1])


# speedup vs baseline: 2.6196x; 1.2076x over previous
import functools

import jax
import jax.numpy as jnp
from jax.experimental import pallas as pl
from jax.experimental.pallas import tpu as pltpu

_NUM_CODES = 1024
_CODE_DIM = 64
_BETA = 0.25


def _vq_kernel(z_ref, cb_ref, zq_ref, idx_ref, sums_ref,
               counts_ref, loss_ref, *, nsteps, total_px):
    b = pl.program_id(0)

    @pl.when(b == 0)
    def _init():
        counts_ref[...] = jnp.zeros_like(counts_ref)
        loss_ref[...] = jnp.zeros_like(loss_ref)

    cb = cb_ref[...]
    e_sq = jnp.sum(cb * cb, axis=1, keepdims=True)
    cbm2 = cb * -2.0

    for i in range(z_ref.shape[0]):
        z_b = z_ref[i]
        z_sq = jnp.sum(z_b * z_b, axis=0, keepdims=True)
        s = jax.lax.dot_general(cbm2, z_b, (((1,), (0,)), ((), ())),
                                preferred_element_type=jnp.float32)
        d = (z_sq + e_sq) + s

        idx = jnp.argmin(d, axis=0, keepdims=True)
        idx_ref[i] = idx

        code_iota = jax.lax.broadcasted_iota(jnp.int32, d.shape, 0)
        onehot = (code_iota == idx).astype(jnp.float32)
        z_q = jax.lax.dot_general(cb, onehot, (((0,), (0,)), ((), ())),
                                  preferred_element_type=jnp.float32)
        zq_ref[i] = z_q

        counts_ref[...] += jnp.sum(onehot, axis=1, keepdims=True)
        r = z_q - z_b
        loss_ref[...] += jnp.sum(r * r, axis=0, keepdims=True)

    @pl.when(b == nsteps - 1)
    def _finalize():
        loss_sum = jnp.sum(loss_ref[...], axis=1, keepdims=True)
        vq_loss = loss_sum * ((1.0 + _BETA) / total_px)
        p = counts_ref[...] * (1.0 / (total_px / _CODE_DIM))
        ent = jnp.sum(p * jnp.log(p + 1e-10), axis=0, keepdims=True)
        perplexity = jnp.exp(-ent)
        sums_ref[0:1, 0:1] = vq_loss
        sums_ref[0:1, 1:2] = perplexity


def kernel(z, codebook):
    B, C, H, W = z.shape
    P = H * W
    BPB = 2
    nsteps = B // BPB
    z_r = z.reshape(B, C, P)
    total_px = B * C * P

    grid = (nsteps,)
    zq3, idx3, sums = pl.pallas_call(
        functools.partial(_vq_kernel, nsteps=nsteps, total_px=total_px),
        grid=grid,
        in_specs=[
            pl.BlockSpec((BPB, C, P), lambda b: (b, 0, 0)),
            pl.BlockSpec((_NUM_CODES, _CODE_DIM), lambda b: (0, 0)),
        ],
        out_specs=[
            pl.BlockSpec((BPB, C, P), lambda b: (b, 0, 0)),
            pl.BlockSpec((BPB, 1, P), lambda b: (b, 0, 0)),
            pl.BlockSpec((8, 128), lambda b: (0, 0)),
        ],
        out_shape=[
            jax.ShapeDtypeStruct((B, C, P), jnp.float32),
            jax.ShapeDtypeStruct((B, 1, P), jnp.int32),
            jax.ShapeDtypeStruct((8, 128), jnp.float32),
        ],
        scratch_shapes=[
            pltpu.VMEM((_NUM_CODES, 1), jnp.float32),
            pltpu.VMEM((1, P), jnp.float32),
        ],
    )(z_r, codebook)

    z_q_st = zq3.reshape(B, C, H, W)
    indices = idx3.reshape(B, H, W)
    return (z_q_st, indices, sums[0, 0], sums[0, 1])
